# Initial kernel scaffold; baseline (speedup 1.0000x reference)
import jax
import jax.numpy as jnp
from jax.experimental import pallas as pl


def kernel(x, edges, node2edges, target_nodes):
    raise NotImplementedError("write your pallas kernel here")


# speedup vs baseline: 2.2023x; 2.2023x over previous
import functools
import math

import jax
import jax.numpy as jnp
from jax import lax
from jax.experimental import pallas as pl
from jax.experimental.pallas import tpu as pltpu
from jax.experimental.pallas import tpu_sc as plsc

NC = 2
NS = 16
NW = NC * NS
L = 16

SA = 64


def _stage_a(n_iters, D, scale):
    def body(x_hbm, et_hbm, em_hbm, idx_v, r0, r1, r2, sum_v, s0, s1, s2):
        wid = lax.axis_index("s") * NC + lax.axis_index("c")
        base = wid * (n_iters * SA)
        pltpu.sync_copy(et_hbm.at[wid], idx_v)

        @pl.loop(0, n_iters)
        def _(j):
            c0 = pltpu.async_copy(x_hbm.at[idx_v.at[0, j]], r0, s0)
            c1 = pltpu.async_copy(x_hbm.at[idx_v.at[1, j]], r1, s1)
            c2 = pltpu.async_copy(x_hbm.at[idx_v.at[2, j]], r2, s2)
            c0.wait()
            c1.wait()
            c2.wait()

            @pl.loop(0, SA)
            def _(row):
                for v in range(D // L):
                    sl = pl.ds(v * L, L)
                    sum_v[row, sl] = (r0[row, sl] + r1[row, sl] + r2[row, sl]) * scale

            pltpu.sync_copy(sum_v, em_hbm.at[pl.ds(base + j * SA, SA)])

    return body


def _stage_b(nodes_per_w, DEG, D):
    def body(em_hbm, tgt_hbm, out_hbm, idx_v, rows, out_buf, s0, s1):
        wid = lax.axis_index("s") * NC + lax.axis_index("c")
        pltpu.sync_copy(tgt_hbm.at[wid], idx_v)

        sems = (s0, s1)
        for b in range(2):
            pltpu.async_copy(em_hbm.at[idx_v.at[b]], rows.at[b], sems[b])

        @pl.loop(0, nodes_per_w, step=2)
        def _(n):
            for b in range(2):
                i = n + b
                pltpu.make_async_copy(em_hbm.at[idx_v.at[i]], rows.at[b], sems[b]).wait()
                for v in range(D // L):
                    sl = pl.ds(v * L, L)
                    acc = rows[b, 0, sl]
                    for k in range(1, DEG):
                        acc = acc + rows[b, k, sl]
                    out_buf[i, sl] = acc
                nxt = i + 2

                @pl.when(nxt < nodes_per_w)
                def _():
                    pltpu.async_copy(em_hbm.at[idx_v.at[nxt]], rows.at[b], sems[b])

        pltpu.sync_copy(out_buf, out_hbm.at[pl.ds(wid * nodes_per_w, nodes_per_w)])

    return body


def kernel(x, edges, node2edges, target_nodes):
    N, D = x.shape
    E, M = edges.shape
    DEG = node2edges.shape[1]
    B = target_nodes.shape[0]
    assert M == 3, "kernel specialized for cardinality-3 hyperedges"

    alpha = sum((-1) ** j * math.comb(M, j) * (M - j) ** M for j in range(M))
    scale = (M / alpha) / DEG * float(math.factorial(M - 1)) / M

    JA = -(-E // (NW * SA))
    E_pad = NW * JA * SA
    et = jnp.pad(edges, ((0, E_pad - E), (0, 0))).T
    et = et.reshape(M, NW, JA, SA).transpose(1, 0, 2, 3)

    stage_a = functools.partial(
        pl.kernel,
        out_type=jax.ShapeDtypeStruct((E_pad, D), jnp.float32),
        mesh=plsc.VectorSubcoreMesh(core_axis_name="c", subcore_axis_name="s"),
        scratch_types=[
            pltpu.VMEM((M, JA, SA), jnp.int32),
            pltpu.VMEM((SA, D), jnp.float32),
            pltpu.VMEM((SA, D), jnp.float32),
            pltpu.VMEM((SA, D), jnp.float32),
            pltpu.VMEM((SA, D), jnp.float32),
            pltpu.SemaphoreType.DMA,
            pltpu.SemaphoreType.DMA,
            pltpu.SemaphoreType.DMA,
        ],
    )(_stage_a(JA, D, scale))
    em = stage_a(x, et)

    BW = -(-B // (NW * 8)) * 8
    B_pad = NW * BW
    tgt = jnp.take(node2edges, target_nodes, axis=0)
    tgt = jnp.pad(tgt, ((0, B_pad - B), (0, 0)))
    tgt = tgt.reshape(NW, BW, DEG)

    stage_b = functools.partial(
        pl.kernel,
        out_type=jax.ShapeDtypeStruct((B_pad, D), jnp.float32),
        mesh=plsc.VectorSubcoreMesh(core_axis_name="c", subcore_axis_name="s"),
        scratch_types=[
            pltpu.VMEM((BW, DEG), jnp.int32),
            pltpu.VMEM((2, DEG, D), jnp.float32),
            pltpu.VMEM((BW, D), jnp.float32),
            pltpu.SemaphoreType.DMA,
            pltpu.SemaphoreType.DMA,
        ],
    )(_stage_b(BW, DEG, D))
    out = stage_b(em, tgt)
    return out[:B]


# speedup vs baseline: 2.4054x; 1.0922x over previous
import functools
import math

import jax
import jax.numpy as jnp
from jax import lax
from jax.experimental import pallas as pl
from jax.experimental.pallas import tpu as pltpu
from jax.experimental.pallas import tpu_sc as plsc

NC = 2
NS = 16
NW = NC * NS
L = 16

SA = 48
NB = 8


def _stage_a(n_iters, D, scale):
    def body(x_hbm, et_hbm, em_hbm, idx_v, r, sum_v, s0, s1):
        wid = lax.axis_index("s") * NC + lax.axis_index("c")
        base = wid * (n_iters * SA)
        pltpu.sync_copy(et_hbm.at[wid], idx_v)
        sems = (s0, s1)

        def issue(j, b):
            for m in range(3):
                pltpu.async_copy(x_hbm.at[idx_v.at[m, j]], r.at[b, m], sems[b])

        issue(0, 0)
        issue(1, 1)

        @pl.loop(0, n_iters, step=2)
        def _(n):
            for b in range(2):
                j = n + b
                for m in range(3):
                    pltpu.make_async_copy(
                        x_hbm.at[idx_v.at[m, j]], r.at[b, m], sems[b]
                    ).wait()

                @pl.loop(0, SA, unroll=8)
                def _(row):
                    for v in range(D // L):
                        sl = pl.ds(v * L, L)
                        sum_v[row, sl] = (
                            r[b, 0, row, sl] + r[b, 1, row, sl] + r[b, 2, row, sl]
                        ) * scale

                pltpu.sync_copy(sum_v, em_hbm.at[pl.ds(base + j * SA, SA)])
                nxt = j + 2

                @pl.when(nxt < n_iters)
                def _():
                    issue(nxt, b)

    return body


def _stage_b(n_groups, DEG, D):
    def body(em_hbm, tgt_hbm, out_hbm, idx_v, rows, ob, g0, g1, o0, o1):
        wid = lax.axis_index("s") * NC + lax.axis_index("c")
        obase = wid * (n_groups * NB)
        pltpu.sync_copy(tgt_hbm.at[wid], idx_v)
        gsems = (g0, g1)
        osems = (o0, o1)

        for b in range(2):
            pltpu.async_copy(em_hbm.at[idx_v.at[b]], rows.at[b], gsems[b])

        @pl.loop(0, n_groups, step=2)
        def _(n):
            for b in range(2):
                g = n + b
                pltpu.make_async_copy(
                    em_hbm.at[idx_v.at[g]], rows.at[b], gsems[b]
                ).wait()

                @pl.when(g >= 2)
                def _():
                    pltpu.make_async_copy(
                        ob.at[b], out_hbm.at[pl.ds(obase + g * NB, NB)], osems[b]
                    ).wait()

                @pl.loop(0, NB)
                def _(m):
                    mb = m * DEG
                    for v in range(D // L):
                        sl = pl.ds(v * L, L)
                        acc = rows[b, mb, sl]
                        for k in range(1, DEG):
                            acc = acc + rows[b, mb + k, sl]
                        ob[b, m, sl] = acc

                pltpu.async_copy(
                    ob.at[b], out_hbm.at[pl.ds(obase + g * NB, NB)], osems[b]
                )
                nxt = g + 2

                @pl.when(nxt < n_groups)
                def _():
                    pltpu.async_copy(em_hbm.at[idx_v.at[nxt]], rows.at[b], gsems[b])

        for b in range(2):
            pltpu.make_async_copy(
                ob.at[b], out_hbm.at[pl.ds(obase, NB)], osems[b]
            ).wait()

    return body


def kernel(x, edges, node2edges, target_nodes):
    N, D = x.shape
    E, M = edges.shape
    DEG = node2edges.shape[1]
    B = target_nodes.shape[0]
    assert M == 3, "kernel specialized for cardinality-3 hyperedges"

    alpha = sum((-1) ** j * math.comb(M, j) * (M - j) ** M for j in range(M))
    scale = (M / alpha) / DEG * float(math.factorial(M - 1)) / M

    JA = -(-E // (NW * SA))
    JA += JA % 2
    E_pad = NW * JA * SA
    et = jnp.pad(edges, ((0, E_pad - E), (0, 0))).T
    et = et.reshape(M, NW, JA, SA).transpose(1, 0, 2, 3)

    stage_a = functools.partial(
        pl.kernel,
        out_type=jax.ShapeDtypeStruct((E_pad, D), jnp.float32),
        mesh=plsc.VectorSubcoreMesh(core_axis_name="c", subcore_axis_name="s"),
        scratch_types=[
            pltpu.VMEM((M, JA, SA), jnp.int32),
            pltpu.VMEM((2, M, SA, D), jnp.float32),
            pltpu.VMEM((SA, D), jnp.float32),
            pltpu.SemaphoreType.DMA,
            pltpu.SemaphoreType.DMA,
        ],
    )(_stage_a(JA, D, scale))
    em = stage_a(x, et)

    NG = -(-B // (NW * NB))
    NG += NG % 2
    B_pad = NW * NG * NB
    tgt = jnp.take(node2edges, target_nodes, axis=0)
    tgt = jnp.pad(tgt, ((0, B_pad - B), (0, 0)))
    tgt = tgt.reshape(NW, NG, NB * DEG)

    stage_b = functools.partial(
        pl.kernel,
        out_type=jax.ShapeDtypeStruct((B_pad, D), jnp.float32),
        mesh=plsc.VectorSubcoreMesh(core_axis_name="c", subcore_axis_name="s"),
        scratch_types=[
            pltpu.VMEM((NG, NB * DEG), jnp.int32),
            pltpu.VMEM((2, NB * DEG, D), jnp.float32),
            pltpu.VMEM((2, NB, D), jnp.float32),
            pltpu.SemaphoreType.DMA,
            pltpu.SemaphoreType.DMA,
            pltpu.SemaphoreType.DMA,
            pltpu.SemaphoreType.DMA,
        ],
    )(_stage_b(NG, DEG, D))
    out = stage_b(em, tgt)
    return out[:B]
